# Initial kernel scaffold; baseline (speedup 1.0000x reference)
import jax
import jax.numpy as jnp
from jax import lax
from jax.experimental import pallas as pl
from jax.experimental.pallas import tpu as pltpu
from jax.experimental.pallas import tpu_sc as plsc

P, S, H, D, T = 512, 16, 8, 128, 2048
NK = S * 2 * H
ROWS = P * NK
NKEY = P * S
NC, NS = 2, 16
NW = NC * NS
KEYS_PER_W = NKEY // NW
ROWS_PER_W = ROWS // NW
NSCAN = T // 16


def _gather16(x, idx):
    dn = lax.GatherDimensionNumbers(
        offset_dims=(), collapsed_slice_dims=(0,), start_index_map=(0,))
    return lax.gather(x, idx.reshape(16, 1), dn, (1,),
                      mode=lax.GatherScatterMode.PROMISE_IN_BOUNDS)


def _body(kv_hbm, tp_hbm, ts_hbm, nk_hbm, nv_hbm, out_hbm,
          tp_v, ts_v, last_v, tokl_v, keyl_v,
          idxs_v, idxdk_v, idxdv_v, bufk_v, bufv_v,
          csem, gksem, gvsem, sksem, svsem):
    wid = lax.axis_index("s") * NC + lax.axis_index("c")
    row0 = wid * ROWS_PER_W

    h_copy = pltpu.async_copy(
        kv_hbm.at[pl.ds(row0, ROWS_PER_W)],
        out_hbm.at[pl.ds(row0, ROWS_PER_W)], csem)

    pltpu.sync_copy(tp_hbm, tp_v)
    pltpu.sync_copy(ts_hbm, ts_v)

    iota = lax.iota(jnp.int32, 16)
    shift_idx = jnp.minimum(iota + 1, 15)

    def init_body(i, _):
        last_v[pl.ds(i * 16, 16)] = jnp.full((16,), -1, jnp.int32)
        return 0
    lax.fori_loop(0, NKEY // 16, init_body, 0)

    def scan_body(i, _):
        base = i * 16
        p = tp_v[pl.ds(base, 16)]
        s = ts_v[pl.ds(base, 16)]
        comb = (p * S + s) * T + base + iota
        comb_s = jnp.sort(comb)
        skey = comb_s >> 11
        stok = comb_s & (T - 1)
        nxt = _gather16(skey, shift_idx)
        run_end = (nxt != skey) | (iota == 15)
        plsc.store_scatter(last_v, [skey], stok, mask=run_end)
        return 0
    lax.fori_loop(0, NSCAN, scan_body, 0)

    key0 = wid * KEYS_PER_W

    def sel_body(c, carry):
        off, best = carry
        kvec = key0 + c * 16 + iota
        wtok = plsc.load_gather(last_v, [kvec])
        m = wtok >= 0
        cnt = jnp.max(plsc.all_reduce_population_count(m))
        plsc.store_compressed(tokl_v.at[pl.ds(off, 16)], wtok, mask=m)
        plsc.store_compressed(keyl_v.at[pl.ds(off, 16)], kvec, mask=m)
        vbest = jnp.max(jnp.where(m, kvec * T + wtok, -1))
        return off + cnt, jnp.maximum(best, vbest)

    w_cnt, best = lax.fori_loop(0, KEYS_PER_W // 16, sel_body,
                                (jnp.int32(0), jnp.int32(-1)))

    @pl.when(w_cnt > 0)
    def _pad():
        tokl_v[pl.ds(w_cnt, 16)] = jnp.full((16,), 1, jnp.int32) * (best & (T - 1))
        keyl_v[pl.ds(w_cnt, 16)] = jnp.full((16,), 1, jnp.int32) * (best >> 11)

    h_copy.wait()

    nchunks = (w_cnt + 15) // 16

    def dma_body(c, _):
        tok16 = tokl_v[pl.ds(c * 16, 16)]
        key16 = keyl_v[pl.ds(c * 16, 16)]
        for j in range(8):
            idxs_v[pl.ds(j * 16, 16)] = tok16 * H + j
            idxdk_v[pl.ds(j * 16, 16)] = key16 * (2 * H) + 2 * j
            idxdv_v[pl.ds(j * 16, 16)] = key16 * (2 * H) + 2 * j + 1
        gk = pltpu.async_copy(nk_hbm.at[idxs_v], bufk_v, gksem)
        gv = pltpu.async_copy(nv_hbm.at[idxs_v], bufv_v, gvsem)
        gk.wait()
        gv.wait()
        sk = pltpu.async_copy(bufk_v, out_hbm.at[idxdk_v], sksem)
        sv = pltpu.async_copy(bufv_v, out_hbm.at[idxdv_v], svsem)
        sk.wait()
        sv.wait()
        return 0
    lax.fori_loop(0, nchunks, dma_body, 0)


@jax.jit
def _kv_scatter(kv_flat, t_pages, t_slots, nk_flat, nv_flat):
    mesh = plsc.VectorSubcoreMesh(core_axis_name="c", subcore_axis_name="s")
    return pl.kernel(
        _body,
        out_type=jax.ShapeDtypeStruct((ROWS, D), jnp.float32),
        mesh=mesh,
        scratch_types=[
            pltpu.VMEM((T,), jnp.int32),
            pltpu.VMEM((T,), jnp.int32),
            pltpu.VMEM((NKEY,), jnp.int32),
            pltpu.VMEM((KEYS_PER_W + 16,), jnp.int32),
            pltpu.VMEM((KEYS_PER_W + 16,), jnp.int32),
            pltpu.VMEM((128,), jnp.int32),
            pltpu.VMEM((128,), jnp.int32),
            pltpu.VMEM((128,), jnp.int32),
            pltpu.VMEM((128, D), jnp.float32),
            pltpu.VMEM((128, D), jnp.float32),
            pltpu.SemaphoreType.DMA,
            pltpu.SemaphoreType.DMA,
            pltpu.SemaphoreType.DMA,
            pltpu.SemaphoreType.DMA,
            pltpu.SemaphoreType.DMA,
        ],
    )(kv_flat, t_pages, t_slots, nk_flat, nv_flat)


def kernel(kv_pages, t_pages, t_slots, new_k, new_v, K):
    del K
    kv_flat = kv_pages.reshape(ROWS, D)
    nk_flat = new_k.astype(jnp.float32).reshape(T * H, D)
    nv_flat = new_v.astype(jnp.float32).reshape(T * H, D)
    out = _kv_scatter(kv_flat, t_pages.astype(jnp.int32),
                      t_slots.astype(jnp.int32), nk_flat, nv_flat)
    return out.reshape(P, S, 2 * H, D)


# speedup vs baseline: 8.4736x; 8.4736x over previous
import jax
import jax.numpy as jnp
from jax import lax
from jax.experimental import pallas as pl
from jax.experimental.pallas import tpu as pltpu
from jax.experimental.pallas import tpu_sc as plsc

P, S, H, D, T = 512, 16, 8, 128, 2048
NK = S * 2 * H
ROWS = P * NK
NKEY = P * S
NC, NS = 2, 16
NW = NC * NS
KEYS_PER_W = NKEY // NW
ROWS_PER_W = ROWS // NW
NSCAN = T // 16


def _gather16(x, idx):
    dn = lax.GatherDimensionNumbers(
        offset_dims=(), collapsed_slice_dims=(0,), start_index_map=(0,))
    return lax.gather(x, idx.reshape(16, 1), dn, (1,),
                      mode=lax.GatherScatterMode.PROMISE_IN_BOUNDS)


def _body(kv_hbm, tp_hbm, ts_hbm, nk_hbm, nv_hbm, out_hbm,
          tp_v, ts_v, last_v, tokl_v, keyl_v,
          idxs_v, idxdk_v, idxdv_v, bufk_v, bufv_v,
          csem, gksem, gvsem, sksem, svsem):
    wid = lax.axis_index("s") * NC + lax.axis_index("c")
    row0 = wid * ROWS_PER_W

    h_copy = pltpu.async_copy(
        kv_hbm.at[pl.ds(row0, ROWS_PER_W)],
        out_hbm.at[pl.ds(row0, ROWS_PER_W)], csem)

    pltpu.sync_copy(tp_hbm, tp_v)
    pltpu.sync_copy(ts_hbm, ts_v)

    iota = lax.iota(jnp.int32, 16)
    shift_idx = jnp.minimum(iota + 1, 15)

    def init_body(i, _):
        last_v[pl.ds(i * 16, 16)] = jnp.full((16,), -1, jnp.int32)
        return 0
    lax.fori_loop(0, NKEY // 16, init_body, 0)

    def scan_body(i, _):
        base = i * 16
        p = tp_v[pl.ds(base, 16)]
        s = ts_v[pl.ds(base, 16)]
        key = p * S + s
        loser = iota < 0
        for d in range(1, 16):
            shifted = _gather16(key, jnp.minimum(iota + d, 15))
            loser = loser | ((key == shifted) & (iota + d <= 15))
        plsc.store_scatter(last_v, [key], base + iota, mask=~loser)
        return 0
    lax.fori_loop(0, NSCAN, scan_body, 0)

    key0 = wid * KEYS_PER_W

    def sel_body(c, carry):
        off, best = carry
        kvec = key0 + c * 16 + iota
        wtok = plsc.load_gather(last_v, [kvec])
        m = wtok >= 0
        cnt = jnp.max(plsc.all_reduce_population_count(m))
        plsc.store_compressed(tokl_v.at[pl.ds(off, 16)], wtok, mask=m)
        plsc.store_compressed(keyl_v.at[pl.ds(off, 16)], kvec, mask=m)
        vbest = jnp.max(jnp.where(m, kvec * T + wtok, -1))
        return off + cnt, jnp.maximum(best, vbest)

    w_cnt, best = lax.fori_loop(0, KEYS_PER_W // 16, sel_body,
                                (jnp.int32(0), jnp.int32(-1)))

    @pl.when(w_cnt > 0)
    def _pad():
        tokl_v[pl.ds(w_cnt, 16)] = jnp.full((16,), 1, jnp.int32) * (best & (T - 1))
        keyl_v[pl.ds(w_cnt, 16)] = jnp.full((16,), 1, jnp.int32) * (best >> 11)

    h_copy.wait()

    nchunks = (w_cnt + 15) // 16

    def dma_body(c, _):
        tok16 = tokl_v[pl.ds(c * 16, 16)]
        key16 = keyl_v[pl.ds(c * 16, 16)]
        for j in range(8):
            idxs_v[pl.ds(j * 16, 16)] = tok16 * H + j
            idxdk_v[pl.ds(j * 16, 16)] = key16 * (2 * H) + 2 * j
            idxdv_v[pl.ds(j * 16, 16)] = key16 * (2 * H) + 2 * j + 1
        gk = pltpu.async_copy(nk_hbm.at[idxs_v], bufk_v, gksem)
        gv = pltpu.async_copy(nv_hbm.at[idxs_v], bufv_v, gvsem)
        gk.wait()
        gv.wait()
        sk = pltpu.async_copy(bufk_v, out_hbm.at[idxdk_v], sksem)
        sv = pltpu.async_copy(bufv_v, out_hbm.at[idxdv_v], svsem)
        sk.wait()
        sv.wait()
        return 0
    lax.fori_loop(0, nchunks, dma_body, 0)


@jax.jit
def _kv_scatter(kv_flat, t_pages, t_slots, nk_flat, nv_flat):
    mesh = plsc.VectorSubcoreMesh(core_axis_name="c", subcore_axis_name="s")
    return pl.kernel(
        _body,
        out_type=jax.ShapeDtypeStruct((ROWS, D), jnp.float32),
        mesh=mesh,
        compiler_params=pltpu.CompilerParams(needs_layout_passes=False),
        scratch_types=[
            pltpu.VMEM((T,), jnp.int32),
            pltpu.VMEM((T,), jnp.int32),
            pltpu.VMEM((NKEY,), jnp.int32),
            pltpu.VMEM((KEYS_PER_W + 16,), jnp.int32),
            pltpu.VMEM((KEYS_PER_W + 16,), jnp.int32),
            pltpu.VMEM((128,), jnp.int32),
            pltpu.VMEM((128,), jnp.int32),
            pltpu.VMEM((128,), jnp.int32),
            pltpu.VMEM((128, D), jnp.float32),
            pltpu.VMEM((128, D), jnp.float32),
            pltpu.SemaphoreType.DMA,
            pltpu.SemaphoreType.DMA,
            pltpu.SemaphoreType.DMA,
            pltpu.SemaphoreType.DMA,
            pltpu.SemaphoreType.DMA,
        ],
    )(kv_flat, t_pages, t_slots, nk_flat, nv_flat)


def kernel(kv_pages, t_pages, t_slots, new_k, new_v, K):
    del K
    kv_flat = kv_pages.reshape(ROWS, D)
    nk_flat = new_k.astype(jnp.float32).reshape(T * H, D)
    nv_flat = new_v.astype(jnp.float32).reshape(T * H, D)
    out = _kv_scatter(kv_flat, t_pages.astype(jnp.int32),
                      t_slots.astype(jnp.int32), nk_flat, nv_flat)
    return out.reshape(P, S, 2 * H, D)


# speedup vs baseline: 184.0004x; 21.7145x over previous
import jax
import jax.numpy as jnp
from jax import lax
from jax.experimental import pallas as pl
from jax.experimental.pallas import tpu as pltpu
from jax.experimental.pallas import tpu_sc as plsc

P, S, H, D, T = 512, 16, 8, 128, 2048
NK = S * 2 * H
ROWS = P * NK
NKEY = P * S
NC, NS = 2, 16
NW = NC * NS
KEYS_PER_W = NKEY // NW
ROWS_PER_W = ROWS // NW
NSCAN = T // 16


def _gather16(x, idx):
    dn = lax.GatherDimensionNumbers(
        offset_dims=(), collapsed_slice_dims=(0,), start_index_map=(0,))
    return lax.gather(x, idx.reshape(16, 1), dn, (1,),
                      mode=lax.GatherScatterMode.PROMISE_IN_BOUNDS)


CROWS = 256
NCH = ROWS_PER_W // CROWS


def _body(kv_hbm, tp_hbm, ts_hbm, nk_hbm, nv_hbm, out_hbm,
          tp_v, ts_v, last_v, tokl_v, keyl_v,
          idxs_v, idxdk_v, idxdv_v, bufk_v, bufv_v, cb0_v, cb1_v,
          cs0, cs1, cs2, cs3, gksem, gvsem, sksem, svsem):
    wid = lax.axis_index("s") * NC + lax.axis_index("c")
    row0 = wid * ROWS_PER_W

    pltpu.sync_copy(tp_hbm, tp_v)
    pltpu.sync_copy(ts_hbm, ts_v)

    iota = lax.iota(jnp.int32, 16)
    shift_idx = jnp.minimum(iota + 1, 15)

    def init_body(i, _):
        last_v[pl.ds(i * 16, 16)] = jnp.full((16,), -1, jnp.int32)
        return 0
    lax.fori_loop(0, NKEY // 16, init_body, 0)

    def scan_body(i, _):
        base = i * 16
        p = tp_v[pl.ds(base, 16)]
        s = ts_v[pl.ds(base, 16)]
        key = p * S + s
        loser = iota < 0
        for d in range(1, 16):
            shifted = _gather16(key, jnp.minimum(iota + d, 15))
            loser = loser | ((key == shifted) & (iota + d <= 15))
        plsc.store_scatter(last_v, [key], base + iota, mask=~loser)
        return 0
    lax.fori_loop(0, NSCAN, scan_body, 0)

    key0 = wid * KEYS_PER_W

    def sel_body(c, carry):
        off, best = carry
        kvec = key0 + c * 16 + iota
        wtok = plsc.load_gather(last_v, [kvec])
        m = wtok >= 0
        cnt = jnp.max(plsc.all_reduce_population_count(m))
        plsc.store_compressed(tokl_v.at[pl.ds(off, 16)], wtok, mask=m)
        plsc.store_compressed(keyl_v.at[pl.ds(off, 16)], kvec, mask=m)
        vbest = jnp.max(jnp.where(m, kvec * T + wtok, -1))
        return off + cnt, jnp.maximum(best, vbest)

    w_cnt, best = lax.fori_loop(0, KEYS_PER_W // 16, sel_body,
                                (jnp.int32(0), jnp.int32(-1)))

    @pl.when(w_cnt > 0)
    def _pad():
        tokl_v[pl.ds(w_cnt, 16)] = jnp.full((16,), 1, jnp.int32) * (best & (T - 1))
        keyl_v[pl.ds(w_cnt, 16)] = jnp.full((16,), 1, jnp.int32) * (best >> 11)

    def copy_body(i, _):
        base = row0 + i * (2 * CROWS)
        r0 = pltpu.async_copy(kv_hbm.at[pl.ds(base, CROWS)], cb0_v, cs0)
        r1 = pltpu.async_copy(kv_hbm.at[pl.ds(base + CROWS, CROWS)], cb1_v, cs1)
        r0.wait()
        w0 = pltpu.async_copy(cb0_v, out_hbm.at[pl.ds(base, CROWS)], cs2)
        r1.wait()
        w1 = pltpu.async_copy(cb1_v, out_hbm.at[pl.ds(base + CROWS, CROWS)], cs3)
        w0.wait()
        w1.wait()
        return 0
    lax.fori_loop(0, NCH // 2, copy_body, 0)

    nchunks = (w_cnt + 15) // 16

    def dma_body(c, _):
        tok16 = tokl_v[pl.ds(c * 16, 16)]
        key16 = keyl_v[pl.ds(c * 16, 16)]
        for j in range(8):
            idxs_v[pl.ds(j * 16, 16)] = tok16 * H + j
            idxdk_v[pl.ds(j * 16, 16)] = key16 * (2 * H) + 2 * j
            idxdv_v[pl.ds(j * 16, 16)] = key16 * (2 * H) + 2 * j + 1
        gk = pltpu.async_copy(nk_hbm.at[idxs_v], bufk_v, gksem)
        gv = pltpu.async_copy(nv_hbm.at[idxs_v], bufv_v, gvsem)
        gk.wait()
        gv.wait()
        sk = pltpu.async_copy(bufk_v, out_hbm.at[idxdk_v], sksem)
        sv = pltpu.async_copy(bufv_v, out_hbm.at[idxdv_v], svsem)
        sk.wait()
        sv.wait()
        return 0
    lax.fori_loop(0, nchunks, dma_body, 0)


@jax.jit
def _kv_scatter(kv_flat, t_pages, t_slots, nk_flat, nv_flat):
    mesh = plsc.VectorSubcoreMesh(core_axis_name="c", subcore_axis_name="s")
    return pl.kernel(
        _body,
        out_type=jax.ShapeDtypeStruct((ROWS, D), jnp.float32),
        mesh=mesh,
        compiler_params=pltpu.CompilerParams(needs_layout_passes=False),
        scratch_types=[
            pltpu.VMEM((T,), jnp.int32),
            pltpu.VMEM((T,), jnp.int32),
            pltpu.VMEM((NKEY,), jnp.int32),
            pltpu.VMEM((KEYS_PER_W + 16,), jnp.int32),
            pltpu.VMEM((KEYS_PER_W + 16,), jnp.int32),
            pltpu.VMEM((128,), jnp.int32),
            pltpu.VMEM((128,), jnp.int32),
            pltpu.VMEM((128,), jnp.int32),
            pltpu.VMEM((128, D), jnp.float32),
            pltpu.VMEM((128, D), jnp.float32),
            pltpu.VMEM((CROWS, D), jnp.float32),
            pltpu.VMEM((CROWS, D), jnp.float32),
            pltpu.SemaphoreType.DMA,
            pltpu.SemaphoreType.DMA,
            pltpu.SemaphoreType.DMA,
            pltpu.SemaphoreType.DMA,
            pltpu.SemaphoreType.DMA,
            pltpu.SemaphoreType.DMA,
            pltpu.SemaphoreType.DMA,
            pltpu.SemaphoreType.DMA,
        ],
    )(kv_flat, t_pages, t_slots, nk_flat, nv_flat)


def kernel(kv_pages, t_pages, t_slots, new_k, new_v, K):
    del K
    kv_flat = kv_pages.reshape(ROWS, D)
    nk_flat = new_k.astype(jnp.float32).reshape(T * H, D)
    nv_flat = new_v.astype(jnp.float32).reshape(T * H, D)
    out = _kv_scatter(kv_flat, t_pages.astype(jnp.int32),
                      t_slots.astype(jnp.int32), nk_flat, nv_flat)
    return out.reshape(P, S, 2 * H, D)
